# baseline (speedup 1.0000x reference)
import jax
import jax.numpy as jnp
from jax.experimental import pallas as pl

_K = 512
_NCOLS = 2048
_ROWS_PER_BLOCK = 256


def _topk_mask_kernel(x_ref, o_ref):
    x = x_ref[...]
    keys = jax.lax.bitcast_convert_type(x, jnp.int32) & jnp.int32(0x7FFFFFFF)
    hi16 = keys >> 15
    hi16p = (hi16 - 32768).astype(jnp.int16)

    prefix_hi = jnp.zeros((x.shape[0], 1), jnp.int32)
    for b in range(15, -1, -1):
        cand = prefix_hi | jnp.int32(1 << b)
        candp = (cand - 32768).astype(jnp.int16)
        cnt = jnp.sum((hi16p >= candp).astype(jnp.int16), axis=1,
                      keepdims=True).astype(jnp.int32)
        prefix_hi = jnp.where(cnt >= _K, cand, prefix_hi)

    prefix_hip = (prefix_hi - 32768).astype(jnp.int16)
    c_hi = jnp.sum((hi16p > prefix_hip).astype(jnp.int16), axis=1,
                   keepdims=True).astype(jnp.int32)
    low15 = (keys & jnp.int32(0x7FFF)).astype(jnp.int16)
    masked_low = jnp.where(hi16p == prefix_hip, low15, jnp.int16(-1))

    prefix_lo = jnp.zeros((x.shape[0], 1), jnp.int32)
    for b in range(14, -1, -1):
        cand = prefix_lo | jnp.int32(1 << b)
        candp = cand.astype(jnp.int16)
        cnt = c_hi + jnp.sum((masked_low >= candp).astype(jnp.int16), axis=1,
                             keepdims=True).astype(jnp.int32)
        prefix_lo = jnp.where(cnt >= _K, cand, prefix_lo)

    thresh = (prefix_hi << 15) | prefix_lo
    cnt = jnp.sum((keys >= thresh).astype(jnp.int32), axis=1, keepdims=True)
    scale = jnp.float32(_NCOLS) / cnt.astype(jnp.float32)
    o_ref[...] = jnp.where(keys >= thresh, x * scale, 0.0)


def kernel(x):
    shape = x.shape
    flat = x.reshape(-1, shape[-1])
    n_rows = flat.shape[0]
    out = pl.pallas_call(
        _topk_mask_kernel,
        grid=(n_rows // _ROWS_PER_BLOCK,),
        in_specs=[pl.BlockSpec((_ROWS_PER_BLOCK, _NCOLS), lambda i: (i, 0))],
        out_specs=pl.BlockSpec((_ROWS_PER_BLOCK, _NCOLS), lambda i: (i, 0)),
        out_shape=jax.ShapeDtypeStruct(flat.shape, flat.dtype),
    )(flat)
    return out.reshape(shape), 0, 0


# speedup vs baseline: 2.3342x; 2.3342x over previous
import jax
import jax.numpy as jnp
from jax.experimental import pallas as pl

_K = 512
_NCOLS = 2048
_ROWS_PER_BLOCK = 256


def _topk_mask_kernel(x_ref, o_ref):
    x = x_ref[...]
    keys = jax.lax.bitcast_convert_type(x, jnp.int32) & jnp.int32(0x7FFFFFFF)
    prefix = jnp.zeros((x.shape[0], 1), jnp.int32)
    for b in range(30, -1, -1):
        cand = prefix | jnp.int32(1 << b)
        cnt = jnp.sum(keys >= cand, axis=1, keepdims=True, dtype=jnp.float32)
        prefix = jnp.where(cnt >= jnp.float32(_K), cand, prefix)
    cnt = jnp.sum((keys >= prefix).astype(jnp.int32), axis=1, keepdims=True)
    scale = jnp.float32(_NCOLS) / cnt.astype(jnp.float32)
    o_ref[...] = jnp.where(keys >= prefix, x * scale, 0.0)


def kernel(x):
    shape = x.shape
    flat = x.reshape(-1, shape[-1])
    n_rows = flat.shape[0]
    out = pl.pallas_call(
        _topk_mask_kernel,
        grid=(n_rows // _ROWS_PER_BLOCK,),
        in_specs=[pl.BlockSpec((_ROWS_PER_BLOCK, _NCOLS), lambda i: (i, 0))],
        out_specs=pl.BlockSpec((_ROWS_PER_BLOCK, _NCOLS), lambda i: (i, 0)),
        out_shape=jax.ShapeDtypeStruct(flat.shape, flat.dtype),
    )(flat)
    return out.reshape(shape), 0, 0
